# baseline (speedup 1.0000x reference)
import functools
import math

import jax
import jax.numpy as jnp
from jax import lax
from jax.experimental import pallas as pl
from jax.experimental.pallas import tpu as pltpu
from jax.experimental.pallas import tpu_sc as plsc

EMBED_DIM = 64
DIM1 = EMBED_DIM // 3
NUM_SAR = 12
NUM_OPT = 2500
NUM_ROWS = NUM_SAR + NUM_OPT

BATCH = 4096
SEQ = 200

NC, NS = 2, 16
NW = NC * NS
BPW = BATCH // NW
CB = 2
ROWS = CB * SEQ
NCHUNK = BPW // CB


def _table_body(t_ref, r_ref, o_ref, out_ref):
    R, C = NUM_ROWS, EMBED_DIM
    r = lax.broadcasted_iota(jnp.int32, (R, C), 0)
    c = lax.broadcasted_iota(jnp.int32, (R, C), 1)
    pos = (r - NUM_SAR).astype(jnp.float32)
    j = (c % 32).astype(jnp.float32)
    omega = jnp.exp(j * (-math.log(10000.0) / 32.0))
    ang = pos * omega
    sincos = jnp.where(c < 32, jnp.sin(ang), jnp.cos(ang))
    s = 11 - r
    sm4 = s % 4
    q = s // 4
    t0 = jnp.broadcast_to(t_ref[0:1, :], (R, C))
    t1 = jnp.broadcast_to(t_ref[1:2, :], (R, C))
    r0 = jnp.broadcast_to(r_ref[0:1, :], (R, C))
    r1 = jnp.broadcast_to(r_ref[1:2, :], (R, C))
    o0 = jnp.broadcast_to(o_ref[0:1, :], (R, C))
    o1 = jnp.broadcast_to(o_ref[1:2, :], (R, C))
    tv = jnp.where(sm4 < 2, t0, t1)
    rv = jnp.where((sm4 == 0) | (sm4 == 3), r0, r1)
    ov = jnp.where(q == 0, 0.5 * (o0 + o1), jnp.where(q == 1, o0, o1))
    sarv = jnp.where(c < DIM1, tv, jnp.where(c < 2 * DIM1, rv, ov))
    out_ref[...] = jnp.where(r < NUM_SAR, sarv, sincos)


def _build_table(embed_transmit, embed_receive, embed_orbit):
    f32 = jnp.float32
    t = jnp.zeros((2, EMBED_DIM), f32).at[:, 0:DIM1].set(embed_transmit)
    r = jnp.zeros((2, EMBED_DIM), f32).at[:, DIM1:2 * DIM1].set(embed_receive)
    o = jnp.zeros((2, EMBED_DIM), f32).at[:, 2 * DIM1:].set(embed_orbit)
    return pl.pallas_call(
        _table_body,
        out_shape=jax.ShapeDtypeStruct((NUM_ROWS, EMBED_DIM), f32),
    )(t, r, o)


_GATHER_SPLITS = [(0, 128), (128, 72), (200, 128), (328, 72)]


@functools.partial(
    pl.kernel,
    out_type=jax.ShapeDtypeStruct((BATCH, SEQ, EMBED_DIM), jnp.float32),
    mesh=plsc.VectorSubcoreMesh(core_axis_name="c", subcore_axis_name="s"),
    scratch_types=[
        pltpu.VMEM((2, ROWS), jnp.int32),
        pltpu.VMEM((2, CB, SEQ, EMBED_DIM), jnp.float32),
        pltpu.SemaphoreType.DMA,
        pltpu.SemaphoreType.DMA,
        pltpu.SemaphoreType.DMA,
        pltpu.SemaphoreType.DMA,
        pltpu.SemaphoreType.DMA,
        pltpu.SemaphoreType.DMA,
    ],
    compiler_params=pltpu.CompilerParams(use_tc_tiling_on_sc=False),
)
def _gather(table_hbm, idx_hbm, out_hbm, idx_v, rows_v,
            sem_i0, sem_i1, sem_g0, sem_g1, sem_o0, sem_o1):
    wid = lax.axis_index("s") * NC + lax.axis_index("c")
    rbase = wid * BPW * SEQ
    bbase = wid * BPW
    sem_i = (sem_i0, sem_i1)
    sem_g = (sem_g0, sem_g1)
    sem_o = (sem_o0, sem_o1)

    pltpu.async_copy(idx_hbm.at[pl.ds(rbase, ROWS)], idx_v.at[0], sem_i[0])

    def body(i, carry):
        for b in range(2):
            c = 2 * i + b
            roff = c * ROWS

            pltpu.make_async_copy(
                idx_hbm.at[pl.ds(rbase + roff, ROWS)], idx_v.at[b], sem_i[b]
            ).wait()

            @pl.when(c + 1 < NCHUNK)
            def _():
                pltpu.async_copy(
                    idx_hbm.at[pl.ds(rbase + roff + ROWS, ROWS)],
                    idx_v.at[1 - b],
                    sem_i[1 - b],
                )

            for k in range(ROWS // 16):
                sl = pl.ds(k * 16, 16)
                idx_v[b, sl] = idx_v[b, sl] + NUM_SAR

            @pl.when(i > 0)
            def _():
                pltpu.make_async_copy(
                    rows_v.at[b],
                    out_hbm.at[pl.ds(bbase + c * CB, CB)],
                    sem_o[b],
                ).wait()

            copies = [
                pltpu.async_copy(
                    table_hbm.at[idx_v.at[b, pl.ds(start, n)]],
                    rows_v.at[b, start // SEQ, pl.ds(start % SEQ, n)],
                    sem_g[b],
                )
                for start, n in _GATHER_SPLITS
            ]
            for cp in copies:
                cp.wait()

            pltpu.async_copy(
                rows_v.at[b], out_hbm.at[pl.ds(bbase + c * CB, CB)], sem_o[b]
            )
        return carry

    lax.fori_loop(0, NCHUNK // 2, body, 0)

    pltpu.make_async_copy(
        rows_v.at[0], out_hbm.at[pl.ds(bbase + (NCHUNK - 2) * CB, CB)], sem_o[0]
    ).wait()
    pltpu.make_async_copy(
        rows_v.at[1], out_hbm.at[pl.ds(bbase + (NCHUNK - 1) * CB, CB)], sem_o[1]
    ).wait()


def kernel(input, embed_transmit, embed_receive, embed_orbit):
    table = _build_table(embed_transmit, embed_receive, embed_orbit)
    idx = input.reshape(-1).astype(jnp.int32)
    return _gather(table, idx)
